# Initial kernel scaffold; baseline (speedup 1.0000x reference)
import jax
import jax.numpy as jnp
from jax.experimental import pallas as pl


def kernel(xyz, params):
    raise NotImplementedError("write your pallas kernel here")


# speedup vs baseline: 2.8191x; 2.8191x over previous
import functools

import jax
import jax.numpy as jnp
import numpy as np
from jax.experimental import pallas as pl
from jax.experimental.pallas import tpu as pltpu

_FP = (
    (1.0, 1.0, 1.0), (1.0, 1.0, -1.0), (1.0, -1.0, 1.0), (1.0, -1.0, -1.0),
    (-1.0, 1.0, 1.0), (-1.0, 1.0, -1.0), (-1.0, -1.0, 1.0),
    (-1.0, -1.0, -1.0), (0.0, 0.0, 0.0),
)
_K = 32


def _layer_body(pts_ref, fx_ref, cen_ref, w1_ref, b1_ref, w2t_ref, b2_ref,
                u1_ref, v1_ref, u2_ref, v2_ref, out_ref, gfx_ref,
                *, N, S_blk, C, radius, mean_center):
    f32 = jnp.float32
    px = pts_ref[0, 0:1, :]
    py = pts_ref[0, 1:2, :]
    pz = pts_ref[0, 2:3, :]
    if mean_center:
        cxr = jnp.mean(px, axis=1, keepdims=True)
        cyr = jnp.mean(py, axis=1, keepdims=True)
        czr = jnp.mean(pz, axis=1, keepdims=True)
    else:
        cxr = cen_ref[0, :, 0:1]
        cyr = cen_ref[0, :, 1:2]
        czr = cen_ref[0, :, 2:3]

    dx = cxr - px
    dy = cyr - py
    dz = czr - pz
    d2 = dx * dx + dy * dy + dz * dz

    iota = jax.lax.broadcasted_iota(jnp.int32, (S_blk, N), 1)
    fx = fx_ref[0]

    def sel_step(i, dcur):
        m = jnp.min(dcur, axis=1, keepdims=True)
        am = jnp.min(jnp.where(dcur == m, iota, N), axis=1, keepdims=True)
        hit = iota == am
        selm = hit.astype(f32)
        g = jax.lax.dot_general(selm, fx, (((1,), (0,)), ((), ())),
                                preferred_element_type=f32)
        gfx_ref[pl.ds(i, 1)] = g[None]
        return jnp.where(hit, f32(jnp.inf), dcur)

    jax.lax.fori_loop(0, _K, sel_step, d2)

    gfx = gfx_ref[...]
    gf = gfx[:, :, :C]
    rx = (gfx[:, :, C:C + 1] - cxr[None]) / radius
    ry = (gfx[:, :, C + 1:C + 2] - cyr[None]) / radius
    rz = (gfx[:, :, C + 2:C + 3] - czr[None]) / radius

    w1 = w1_ref[...]
    b1 = b1_ref[...][None]
    w2t = w2t_ref[...][None]
    b2 = b2_ref[0, 0]

    g_best = None
    for f in range(9):
        fpx, fpy, fpz = _FP[f]
        ddx = rx - fpx
        ddy = ry - fpy
        ddz = rz - fpz
        dist = jnp.sqrt(ddx * ddx + ddy * ddy + ddz * ddz)
        a = (ddx * w1_ref[0:1, :][None] + ddy * w1_ref[1:2, :][None]
             + ddz * w1_ref[2:3, :][None] + dist * w1_ref[3:4, :][None]
             + rx * w1_ref[4:5, :][None] + ry * w1_ref[5:6, :][None]
             + rz * w1_ref[6:7, :][None] + b1)
        a = jnp.maximum(a, 0.0)
        wf = jnp.sum(a * w2t, axis=2, keepdims=True) + b2
        wmax = jnp.max(wf, axis=0, keepdims=True)
        e = jnp.exp(wf - wmax)
        wsm = e / jnp.sum(e, axis=0, keepdims=True)
        hf = jnp.sum(wsm * gf, axis=0)
        g_best = hf if g_best is None else jnp.maximum(g_best, hf)

    h1 = jax.lax.dot_general(g_best, u1_ref[...], (((1,), (0,)), ((), ())),
                             preferred_element_type=f32) + v1_ref[...]
    h1 = jnp.maximum(h1, 0.0)
    h2 = jax.lax.dot_general(h1, u2_ref[...], (((1,), (0,)), ((), ())),
                             preferred_element_type=f32) + v2_ref[...]
    out_ref[0] = jnp.maximum(h2, 0.0)


def _run_layer(pts_r, feat, p, npoint, radius, S_blk):
    B, N, _ = pts_r.shape
    C = feat.shape[-1]
    m1h = p['m1W1'].shape[1]
    m2h = p['m2W1'].shape[1]
    cout = p['m2W2'].shape[1]
    if npoint is None:
        S = 1
        cen_r = pts_r[:, :1, :]
        mean_center = True
    else:
        stride = max(N // npoint, 1)
        S = npoint
        cen_r = pts_r[:, ::stride, :]
        mean_center = False

    pts_t = jnp.transpose(pts_r, (0, 2, 1))
    featx = jnp.concatenate([feat, pts_r], axis=-1)
    w1 = p['m1W1']
    b1 = p['m1b1'][None, :]
    w2t = jnp.transpose(p['m1W2'], (1, 0))
    b2 = p['m1b2'][None, :]
    u1 = p['m2W1']
    v1 = p['m2b1'][None, :]
    u2 = p['m2W2']
    v2 = p['m2b2'][None, :]

    body = functools.partial(_layer_body, N=N, S_blk=S_blk, C=C,
                             radius=radius, mean_center=mean_center)
    out = pl.pallas_call(
        body,
        grid=(B, S // S_blk),
        in_specs=[
            pl.BlockSpec((1, 3, N), lambda b, j: (b, 0, 0)),
            pl.BlockSpec((1, N, C + 3), lambda b, j: (b, 0, 0)),
            pl.BlockSpec((1, S_blk, 3), lambda b, j: (b, j, 0)),
            pl.BlockSpec((7, m1h), lambda b, j: (0, 0)),
            pl.BlockSpec((1, m1h), lambda b, j: (0, 0)),
            pl.BlockSpec((1, m1h), lambda b, j: (0, 0)),
            pl.BlockSpec((1, 1), lambda b, j: (0, 0)),
            pl.BlockSpec((C, m2h), lambda b, j: (0, 0)),
            pl.BlockSpec((1, m2h), lambda b, j: (0, 0)),
            pl.BlockSpec((m2h, cout), lambda b, j: (0, 0)),
            pl.BlockSpec((1, cout), lambda b, j: (0, 0)),
        ],
        out_specs=pl.BlockSpec((1, S_blk, cout), lambda b, j: (b, j, 0)),
        out_shape=jax.ShapeDtypeStruct((B, S, cout), jnp.float32),
        scratch_shapes=[pltpu.VMEM((_K, S_blk, C + 3), jnp.float32)],
    )(pts_t, featx, cen_r, w1, b1, w2t, b2, u1, v1, u2, v2)
    return out, cen_r


def _head_body(x_ref, w1_ref, b1_ref, w2_ref, b2_ref, w3_ref, b3_ref, o_ref):
    f32 = jnp.float32
    s = jnp.sqrt(f32(1.0 + 1e-5))
    x = x_ref[...]
    h = jax.lax.dot_general(x, w1_ref[...], (((1,), (0,)), ((), ())),
                            preferred_element_type=f32) + b1_ref[...]
    h = jnp.maximum(h / s, 0.0)
    h = jax.lax.dot_general(h, w2_ref[...], (((1,), (0,)), ((), ())),
                            preferred_element_type=f32) + b2_ref[...]
    h = jnp.maximum(h / s, 0.0)
    o_ref[...] = jax.lax.dot_general(h, w3_ref[...], (((1,), (0,)), ((), ())),
                                     preferred_element_type=f32) + b3_ref[...]


def kernel(xyz, params):
    xyz_r = jnp.transpose(xyz[:, :3, :], (0, 2, 1))
    norm_r = jnp.transpose(xyz[:, 3:, :], (0, 2, 1))
    B = xyz_r.shape[0]

    f1, s1 = _run_layer(xyz_r, norm_r, params['c1'], 512, 0.15, 128)
    f2, s2 = _run_layer(s1, f1, params['c2'], 256, 0.25, 128)
    f3, s3 = _run_layer(s2, f2, params['c3'], 128, 0.4, 128)
    f4, s4 = _run_layer(s3, f3, params['c4'], 32, 0.6, 32)
    f5, _ = _run_layer(s4, f4, params['c5'], None, 10.0, 1)

    x = f5.reshape(B, 1024)
    out = pl.pallas_call(
        _head_body,
        out_shape=jax.ShapeDtypeStruct((B, 40), jnp.float32),
    )(x, params['fc1W'], params['fc1b'][None, :],
      params['fc2W'], params['fc2b'][None, :],
      params['fc3W'], params['fc3b'][None, :])
    return out, s3
